# Initial kernel scaffold; baseline (speedup 1.0000x reference)
import jax
import jax.numpy as jnp
from jax.experimental import pallas as pl


def kernel(X, W, eps, vertex, edges):
    raise NotImplementedError("write your pallas kernel here")


# speedup vs baseline: 1.4801x; 1.4801x over previous
import jax
import jax.numpy as jnp
from jax import lax
from jax.experimental import pallas as pl
from jax.experimental.pallas import tpu as pltpu


def _round_up(x, m):
    return ((x + m - 1) // m) * m


_VMEM_LIMIT = 48 * 1024 * 1024


def _xw_kernel(x_ref, w_ref, o_ref):
    o_ref[...] = jnp.dot(x_ref[...], w_ref[...],
                         preferred_element_type=jnp.float32
                         ).astype(o_ref.dtype)


def _xe_kernel(h_ref, xw_ref, xe_ref, acc_ref):
    k = pl.program_id(1)

    @pl.when(k == 0)
    def _():
        acc_ref[...] = jnp.zeros_like(acc_ref)

    acc_ref[...] += lax.dot_general(
        h_ref[...], xw_ref[...], (((0,), (0,)), ((), ())),
        preferred_element_type=jnp.float32)

    @pl.when(k == pl.num_programs(1) - 1)
    def _():
        xe_ref[...] = acc_ref[...].astype(xe_ref.dtype)


def _out_kernel(eps_ref, xw_ref, h_ref, xe_ref, o_ref, acc_ref):
    e = pl.program_id(1)

    @pl.when(e == 0)
    def _():
        acc_ref[...] = (1.0 + eps_ref[0]) * xw_ref[...].astype(jnp.float32)

    acc_ref[...] += jnp.dot(h_ref[...], xe_ref[...],
                            preferred_element_type=jnp.float32)

    @pl.when(e == pl.num_programs(1) - 1)
    def _():
        o_ref[...] = acc_ref[...]


def kernel(X, W, eps, vertex, edges):
    N, F_in = X.shape
    F = W.shape[1]
    E = 4096

    F_in_p = _round_up(max(F_in, 128), 128)
    Fp = _round_up(max(F, 128), 128)
    Np = _round_up(max(N, 128), 128)
    Ep = _round_up(max(E, 128), 128)

    tn = 256 if Np % 256 == 0 else 128
    te = 512 if Ep % 512 == 0 else 128
    n_n = Np // tn
    n_e = Ep // te

    Hb = jnp.zeros((Np, Ep), jnp.bfloat16).at[vertex, edges].add(1.0)
    Xb = jnp.zeros((Np, F_in_p), jnp.bfloat16).at[:N, :F_in].set(
        X.astype(jnp.bfloat16))
    Wb = jnp.zeros((F_in_p, Fp), jnp.bfloat16).at[:F_in, :F].set(
        W.astype(jnp.bfloat16))
    eps_arr = jnp.asarray(eps, jnp.float32).reshape((1,))

    xw = pl.pallas_call(
        _xw_kernel,
        out_shape=jax.ShapeDtypeStruct((Np, Fp), jnp.bfloat16),
        grid=(n_n,),
        in_specs=[
            pl.BlockSpec((tn, F_in_p), lambda i: (i, 0)),
            pl.BlockSpec((F_in_p, Fp), lambda i: (0, 0)),
        ],
        out_specs=pl.BlockSpec((tn, Fp), lambda i: (i, 0)),
        compiler_params=pltpu.CompilerParams(
            dimension_semantics=("parallel",),
            vmem_limit_bytes=_VMEM_LIMIT,
        ),
    )(Xb, Wb)

    xe = pl.pallas_call(
        _xe_kernel,
        out_shape=jax.ShapeDtypeStruct((Ep, Fp), jnp.bfloat16),
        grid=(n_e, n_n),
        in_specs=[
            pl.BlockSpec((tn, te), lambda e, k: (k, e)),
            pl.BlockSpec((tn, Fp), lambda e, k: (k, 0)),
        ],
        out_specs=pl.BlockSpec((te, Fp), lambda e, k: (e, 0)),
        scratch_shapes=[pltpu.VMEM((te, Fp), jnp.float32)],
        compiler_params=pltpu.CompilerParams(
            dimension_semantics=("parallel", "arbitrary"),
            vmem_limit_bytes=_VMEM_LIMIT,
        ),
    )(Hb, xw)

    out = pl.pallas_call(
        _out_kernel,
        out_shape=jax.ShapeDtypeStruct((Np, Fp), jnp.float32),
        grid=(n_n, n_e),
        in_specs=[
            pl.BlockSpec(memory_space=pltpu.MemorySpace.SMEM),
            pl.BlockSpec((tn, Fp), lambda i, e: (i, 0)),
            pl.BlockSpec((tn, te), lambda i, e: (i, e)),
            pl.BlockSpec((te, Fp), lambda i, e: (e, 0)),
        ],
        out_specs=pl.BlockSpec((tn, Fp), lambda i, e: (i, 0)),
        scratch_shapes=[pltpu.VMEM((tn, Fp), jnp.float32)],
        compiler_params=pltpu.CompilerParams(
            dimension_semantics=("parallel", "arbitrary"),
            vmem_limit_bytes=_VMEM_LIMIT,
        ),
    )(eps_arr, xw, Hb, xe)

    return out[:N, :F]


# speedup vs baseline: 2.2419x; 1.5147x over previous
import jax
import jax.numpy as jnp
from jax import lax
from jax.experimental import pallas as pl
from jax.experimental.pallas import tpu as pltpu


def _round_up(x, m):
    return ((x + m - 1) // m) * m


_VMEM_LIMIT = 100 * 1024 * 1024


def _xw_kernel(x_ref, w_ref, o_ref):
    o_ref[...] = jnp.dot(x_ref[...], w_ref[...],
                         preferred_element_type=jnp.float32
                         ).astype(o_ref.dtype)


def _xe_kernel(h_ref, xw_ref, xe_ref):
    xe_ref[...] = lax.dot_general(
        h_ref[...], xw_ref[...], (((0,), (0,)), ((), ())),
        preferred_element_type=jnp.float32).astype(xe_ref.dtype)


def _out_kernel(eps_ref, xw_ref, h_ref, xe_ref, o_ref):
    o_ref[...] = (1.0 + eps_ref[0]) * xw_ref[...].astype(jnp.float32)
    o_ref[...] += jnp.dot(h_ref[...], xe_ref[...],
                          preferred_element_type=jnp.float32)


def kernel(X, W, eps, vertex, edges):
    N, F_in = X.shape
    F = W.shape[1]
    E = 4096

    F_in_p = _round_up(max(F_in, 128), 128)
    Fp = _round_up(max(F, 128), 128)
    Np = _round_up(max(N, 128), 128)
    Ep = _round_up(max(E, 128), 128)

    tn = 256 if Np % 256 == 0 else 128
    te = 512 if Ep % 512 == 0 else 128
    n_n = Np // tn
    n_e = Ep // te

    Hb = jnp.zeros((Np, Ep), jnp.bfloat16).at[vertex, edges].add(1.0)
    Xb = jnp.zeros((Np, F_in_p), jnp.bfloat16).at[:N, :F_in].set(
        X.astype(jnp.bfloat16))
    Wb = jnp.zeros((F_in_p, Fp), jnp.bfloat16).at[:F_in, :F].set(
        W.astype(jnp.bfloat16))
    eps_arr = jnp.asarray(eps, jnp.float32).reshape((1,))

    xw = pl.pallas_call(
        _xw_kernel,
        out_shape=jax.ShapeDtypeStruct((Np, Fp), jnp.bfloat16),
        grid=(n_n,),
        in_specs=[
            pl.BlockSpec((tn, F_in_p), lambda i: (i, 0)),
            pl.BlockSpec((F_in_p, Fp), lambda i: (0, 0)),
        ],
        out_specs=pl.BlockSpec((tn, Fp), lambda i: (i, 0)),
        compiler_params=pltpu.CompilerParams(
            dimension_semantics=("parallel",),
            vmem_limit_bytes=_VMEM_LIMIT,
        ),
    )(Xb, Wb)

    xe = pl.pallas_call(
        _xe_kernel,
        out_shape=jax.ShapeDtypeStruct((Ep, Fp), jnp.bfloat16),
        grid=(n_e,),
        in_specs=[
            pl.BlockSpec((Np, te), lambda e: (0, e)),
            pl.BlockSpec((Np, Fp), lambda e: (0, 0)),
        ],
        out_specs=pl.BlockSpec((te, Fp), lambda e: (e, 0)),
        compiler_params=pltpu.CompilerParams(
            dimension_semantics=("parallel",),
            vmem_limit_bytes=_VMEM_LIMIT,
        ),
    )(Hb, xw)

    out = pl.pallas_call(
        _out_kernel,
        out_shape=jax.ShapeDtypeStruct((Np, Fp), jnp.float32),
        grid=(n_n,),
        in_specs=[
            pl.BlockSpec(memory_space=pltpu.MemorySpace.SMEM),
            pl.BlockSpec((tn, Fp), lambda i: (i, 0)),
            pl.BlockSpec((tn, Ep), lambda i: (i, 0)),
            pl.BlockSpec((Ep, Fp), lambda i: (0, 0)),
        ],
        out_specs=pl.BlockSpec((tn, Fp), lambda i: (i, 0)),
        compiler_params=pltpu.CompilerParams(
            dimension_semantics=("parallel",),
            vmem_limit_bytes=_VMEM_LIMIT,
        ),
    )(eps_arr, xw, Hb, xe)

    return out[:N, :F]
